# Initial kernel scaffold; baseline (speedup 1.0000x reference)
import jax
import jax.numpy as jnp
from jax.experimental import pallas as pl


def kernel(x, edge_idx, edge_attr, W1, b1, W2, b2):
    raise NotImplementedError("write your pallas kernel here")


# speedup vs baseline: 12.0159x; 12.0159x over previous
import functools

import jax
import jax.numpy as jnp
from jax import lax
from jax.experimental import pallas as pl
from jax.experimental.pallas import tpu as pltpu
from jax.experimental.pallas import tpu_sc as plsc

NC = 2
NS = 16
NW = NC * NS
LANES = 16
K = 128


def _splat_lane(vec, l):
    idx = jnp.full((LANES, 1), l, jnp.int32)
    dnums = lax.GatherDimensionNumbers(
        offset_dims=(), collapsed_slice_dims=(0,), start_index_map=(0,))
    return lax.gather(vec, idx, dnums, slice_sizes=(1,),
                      mode=lax.GatherScatterMode.PROMISE_IN_BOUNDS)


def _sc_mesh():
    return plsc.VectorSubcoreMesh(core_axis_name="c", subcore_axis_name="s",
                                  num_cores=NC, num_subcores=NS)


def _deg_partials(dst3, ew3, n_pad, c_per_tile):
    @functools.partial(
        pl.kernel,
        out_type=jax.ShapeDtypeStruct((NC, n_pad), jnp.float32),
        mesh=_sc_mesh(),
        scratch_types=[
            pltpu.VMEM((c_per_tile, K), jnp.int32),
            pltpu.VMEM((c_per_tile, K), jnp.float32),
            pltpu.VMEM_SHARED((n_pad,), jnp.float32),
        ],
    )
    def deg_kernel(dst_hbm, ew_hbm, out_hbm, dst_v, ew_v, acc):
        cid = lax.axis_index("c")
        sid = lax.axis_index("s")
        wid = cid * NS + sid
        sl_per_tile = n_pad // NS
        base = sid * sl_per_tile

        zeros16 = jnp.zeros((LANES,), jnp.float32)

        @pl.loop(0, K // LANES)
        def _(i):
            ew_v[0, pl.ds(i * LANES, LANES)] = zeros16

        @pl.loop(0, sl_per_tile // K)
        def _(t):
            pltpu.sync_copy(ew_v.at[0], acc.at[pl.ds(base + t * K, K)])

        plsc.subcore_barrier()

        pltpu.sync_copy(dst_hbm.at[wid], dst_v)
        pltpu.sync_copy(ew_hbm.at[wid], ew_v)

        @pl.loop(0, c_per_tile)
        def _(c):
            pltpu.sync_copy(ew_v.at[c], acc.at[dst_v.at[c]], add=True)

        plsc.subcore_barrier()
        pltpu.sync_copy(acc.at[pl.ds(base, sl_per_tile)],
                        out_hbm.at[cid, pl.ds(base, sl_per_tile)])

    return deg_kernel(dst3, ew3)


def _aggregate(hp, src3, dst3, ew3, n_pad, d, c_per_tile):
    @functools.partial(
        pl.kernel,
        out_type=jax.ShapeDtypeStruct((NC, n_pad, d), jnp.float32),
        mesh=_sc_mesh(),
        scratch_types=[
            pltpu.VMEM((c_per_tile, K), jnp.int32),
            pltpu.VMEM((c_per_tile, K), jnp.int32),
            pltpu.VMEM((c_per_tile, K), jnp.float32),
            pltpu.VMEM((K, d), jnp.float32),
            pltpu.VMEM_SHARED((n_pad, d), jnp.float32),
            pltpu.SemaphoreType.DMA,
        ],
    )
    def agg_kernel(hp_hbm, src_hbm, dst_hbm, ew_hbm, out_hbm,
                   src_v, dst_v, ew_v, rows_v, acc, sem):
        cid = lax.axis_index("c")
        sid = lax.axis_index("s")
        wid = cid * NS + sid
        rows_per_tile = n_pad // NS
        base = sid * rows_per_tile
        zeros16 = jnp.zeros((LANES,), jnp.float32)

        @pl.loop(0, K)
        def _(e):
            for j in range(d // LANES):
                rows_v[e, pl.ds(j * LANES, LANES)] = zeros16

        @pl.loop(0, rows_per_tile // K)
        def _(t):
            pltpu.sync_copy(rows_v, acc.at[pl.ds(base + t * K, K)])

        rem = rows_per_tile % K
        if rem:
            pltpu.sync_copy(rows_v.at[pl.ds(0, rem)],
                            acc.at[pl.ds(base + (rows_per_tile // K) * K, rem)])

        plsc.subcore_barrier()

        pltpu.sync_copy(src_hbm.at[wid], src_v)
        pltpu.sync_copy(dst_hbm.at[wid], dst_v)
        pltpu.sync_copy(ew_hbm.at[wid], ew_v)

        @pl.loop(0, c_per_tile)
        def _(c):
            pltpu.async_copy(hp_hbm.at[src_v.at[c]], rows_v, sem).wait()

            @pl.loop(0, K // LANES)
            def _(i):
                ew_vec = ew_v[c, pl.ds(i * LANES, LANES)]
                for l in range(LANES):
                    s = _splat_lane(ew_vec, l)
                    e = i * LANES + l
                    for j in range(d // LANES):
                        sl = pl.ds(j * LANES, LANES)
                        rows_v[e, sl] = rows_v[e, sl] * s

            pltpu.sync_copy(rows_v, acc.at[dst_v.at[c]], add=True)

        plsc.subcore_barrier()
        pltpu.sync_copy(acc.at[pl.ds(base, rows_per_tile)],
                        out_hbm.at[cid, pl.ds(base, rows_per_tile)])

    return agg_kernel(hp, src3, dst3, ew3)


_BLK = 1000


def _dinv_block(degp_ref):
    deg = degp_ref[:, 0] + degp_ref[:, 1] + 1.0
    return lax.rsqrt(deg)[:, None]


def _prep_body(x_ref, w_ref, degp_ref, hp_ref):
    h = jnp.dot(x_ref[...], w_ref[...], preferred_element_type=jnp.float32)
    hp_ref[...] = h * _dinv_block(degp_ref)


def _mid_body(sp_ref, hp_ref, degp_ref, b_ref, w_ref, hp2_ref):
    dinv = _dinv_block(degp_ref)
    s = sp_ref[0] + sp_ref[1] + hp_ref[...]
    out1 = jnp.maximum(dinv * s + b_ref[...], 0.0)
    h2 = jnp.dot(out1, w_ref[...], preferred_element_type=jnp.float32)
    hp2_ref[...] = h2 * dinv


def _final_body(sp_ref, hp_ref, degp_ref, b_ref, out_ref):
    dinv = _dinv_block(degp_ref)
    s = sp_ref[0] + sp_ref[1] + hp_ref[...]
    out_ref[...] = jnp.maximum(dinv * s + b_ref[...], 0.0)


def _row_grid(n, d):
    grid = n // _BLK
    nd_spec = pl.BlockSpec((_BLK, d), lambda i: (i, 0))
    p_spec = pl.BlockSpec((2, _BLK, d), lambda i: (0, i, 0))
    deg_spec = pl.BlockSpec((_BLK, 2), lambda i: (i, 0))
    w_spec = pl.BlockSpec((d, d), lambda i: (0, 0))
    b_spec = pl.BlockSpec((1, d), lambda i: (0, 0))
    return grid, nd_spec, p_spec, deg_spec, w_spec, b_spec


def kernel(x, edge_idx, edge_attr, W1, b1, W2, b2):
    n, d = x.shape
    e = edge_attr.shape[0]

    chunk_all = NW * K
    e_pad = ((e + chunk_all - 1) // chunk_all) * chunk_all
    c_per_tile = e_pad // chunk_all
    n_pad = ((n + NS * K - 1) // (NS * K)) * (NS * K)

    pad = e_pad - e
    src3 = jnp.pad(edge_idx[0], (0, pad)).reshape(NW, c_per_tile, K)
    dst3 = jnp.pad(edge_idx[1], (0, pad)).reshape(NW, c_per_tile, K)
    ew3 = jnp.pad(edge_attr, (0, pad)).reshape(NW, c_per_tile, K)

    degp = _deg_partials(dst3, ew3, n_pad, c_per_tile)[:, :n].T

    grid, nd_spec, p_spec, deg_spec, w_spec, b_spec = _row_grid(n, d)
    out_nd = jax.ShapeDtypeStruct((n, d), jnp.float32)

    hp1 = pl.pallas_call(
        _prep_body, grid=grid,
        in_specs=[nd_spec, w_spec, deg_spec],
        out_specs=nd_spec, out_shape=out_nd,
    )(x, W1, degp)

    s1 = _aggregate(hp1, src3, dst3, ew3, n_pad, d, c_per_tile)

    hp2 = pl.pallas_call(
        _mid_body, grid=grid,
        in_specs=[p_spec, nd_spec, deg_spec, b_spec, w_spec],
        out_specs=nd_spec, out_shape=out_nd,
    )(s1, hp1, degp, b1.reshape(1, d), W2)

    s2 = _aggregate(hp2, src3, dst3, ew3, n_pad, d, c_per_tile)

    out = pl.pallas_call(
        _final_body, grid=grid,
        in_specs=[p_spec, nd_spec, deg_spec, b_spec],
        out_specs=nd_spec, out_shape=out_nd,
    )(s2, hp2, degp, b2.reshape(1, d))

    return out
